# baseline (speedup 1.0000x reference)
import functools

import jax
import jax.numpy as jnp
from jax import lax
from jax.experimental import pallas as pl
from jax.experimental.pallas import tpu as pltpu
from jax.experimental.pallas import tpu_sc as plsc

_NEG = -1e30


def _sc_gather(emb, idx):
    (N,) = idx.shape
    _, D = emb.shape
    info = plsc.get_sparse_core_info()
    nw = info.num_cores * info.num_subcores
    ch = 128
    per_w = N // nw
    n_ch = per_w // ch
    mesh = plsc.VectorSubcoreMesh(core_axis_name="c", subcore_axis_name="s")

    @functools.partial(
        pl.kernel,
        mesh=mesh,
        compiler_params=pltpu.CompilerParams(use_tc_tiling_on_sc=False),
        out_type=jax.ShapeDtypeStruct((N, D), jnp.float32),
        scratch_types=[
            pltpu.VMEM((ch,), jnp.int32),
            pltpu.VMEM((ch, D), jnp.float32),
            pltpu.SemaphoreType.DMA,
        ],
    )
    def gk(emb_hbm, idx_hbm, out_hbm, idx_v, rows_v, sem):
        wid = lax.axis_index("s") * info.num_cores + lax.axis_index("c")
        base = wid * per_w

        def body(i, carry):
            off = base + i * ch
            pltpu.sync_copy(idx_hbm.at[pl.ds(off, ch)], idx_v)
            pltpu.async_copy(emb_hbm.at[idx_v], rows_v, sem).wait()
            pltpu.sync_copy(rows_v, out_hbm.at[pl.ds(off, ch)])
            return carry

        lax.fori_loop(0, n_ch, body, 0)

    return gk(emb, idx)


def _mlp(x, W1, b1, W2, b2, W3, b3, W4, b4):
    Bn, K = x.shape
    Hn = W1.shape[1]
    RB = 512

    def body(x_ref, w1r, b1r, w2r, b2r, w3r, b3r, w4r, b4r, o_ref):
        h = x_ref[...].astype(jnp.bfloat16)
        for w_ref, b_ref in ((w1r, b1r), (w2r, b2r), (w3r, b3r), (w4r, b4r)):
            z = jnp.dot(h, w_ref[...].astype(jnp.bfloat16),
                        preferred_element_type=jnp.float32)
            h = jnp.maximum(z + b_ref[...], 0.0).astype(jnp.bfloat16)
        o_ref[...] = h

    return pl.pallas_call(
        body,
        grid=(Bn // RB,),
        in_specs=[
            pl.BlockSpec((RB, K), lambda i: (i, 0)),
            pl.BlockSpec((K, Hn), lambda i: (0, 0)),
            pl.BlockSpec((1, Hn), lambda i: (0, 0)),
            pl.BlockSpec((Hn, Hn), lambda i: (0, 0)),
            pl.BlockSpec((1, Hn), lambda i: (0, 0)),
            pl.BlockSpec((Hn, Hn), lambda i: (0, 0)),
            pl.BlockSpec((1, Hn), lambda i: (0, 0)),
            pl.BlockSpec((Hn, Hn), lambda i: (0, 0)),
            pl.BlockSpec((1, Hn), lambda i: (0, 0)),
        ],
        out_specs=pl.BlockSpec((RB, Hn), lambda i: (i, 0)),
        out_shape=jax.ShapeDtypeStruct((Bn, Hn), jnp.bfloat16),
    )(x, W1, b1, W2, b2, W3, b3, W4, b4)


_TV = 1024
_ACC = 512
_BT = 1024


def _logits_lse(h, W5, b5):
    Bn, Hn = h.shape
    V = W5.shape[1]
    nv = pl.cdiv(V, _TV)
    bt = min(_BT, Bn)
    nb = Bn // bt

    def body(h_ref, w_ref, b_ref, z_ref, lse_ref, m_ref, acc_ref):
        v = pl.program_id(0)
        b = pl.program_id(1)
        rows = pl.ds(b * bt, bt)
        logits = jnp.dot(h_ref[...], w_ref[...].astype(jnp.bfloat16),
                         preferred_element_type=jnp.float32) + b_ref[...]
        z_ref[...] = logits

        @pl.when(v == 0)
        def _():
            m_ref[rows, :] = jnp.max(logits, axis=1, keepdims=True)
            e = jnp.exp(logits - m_ref[rows, :])
            acc_ref[rows, :] = e[:, :_ACC] + e[:, _ACC:]

        @pl.when(jnp.logical_and(v > 0, v < nv - 1))
        def _():
            e = jnp.exp(logits - m_ref[rows, :])
            acc_ref[rows, :] = acc_ref[rows, :] + e[:, :_ACC] + e[:, _ACC:]

        @pl.when(v == nv - 1)
        def _():
            cols = v * _TV + lax.broadcasted_iota(jnp.int32, (1, _TV), 1)
            e = jnp.exp(jnp.where(cols < V, logits - m_ref[rows, :], _NEG))
            acc = acc_ref[rows, :] + e[:, :_ACC] + e[:, _ACC:]
            lse_ref[...] = m_ref[rows, :] + jnp.log(
                jnp.sum(acc, axis=1, keepdims=True))

    return pl.pallas_call(
        body,
        grid=(nv, nb),
        in_specs=[
            pl.BlockSpec((bt, Hn), lambda v, b: (b, 0)),
            pl.BlockSpec((Hn, _TV), lambda v, b: (0, v)),
            pl.BlockSpec((1, _TV), lambda v, b: (0, v)),
        ],
        out_specs=[
            pl.BlockSpec((bt, _TV), lambda v, b: (b, v)),
            pl.BlockSpec((bt, 1), lambda v, b: (b, 0)),
        ],
        out_shape=[
            jax.ShapeDtypeStruct((Bn, nv * _TV), jnp.float32),
            jax.ShapeDtypeStruct((Bn, 1), jnp.float32),
        ],
        scratch_shapes=[
            pltpu.VMEM((Bn, 1), jnp.float32),
            pltpu.VMEM((Bn, _ACC), jnp.float32),
        ],
    )(h, W5, b5)


def kernel(context_idxs, emb, W1, b1, W2, b2, W3, b3, W4, b4, W5, b5):
    Bn, C = context_idxs.shape
    _, D = emb.shape
    idx = context_idxs.reshape(-1).astype(jnp.int32)
    gathered = _sc_gather(emb, idx)
    x = gathered.reshape(Bn, C * D)
    h = _mlp(x, W1, b1.reshape(1, -1), W2, b2.reshape(1, -1),
             W3, b3.reshape(1, -1), W4, b4.reshape(1, -1))
    b5r = b5.reshape(1, -1)
    z_pad, lse = _logits_lse(h, W5, b5r)
    V = W5.shape[1]
    return z_pad[:, :V] - lse


# speedup vs baseline: 1.0012x; 1.0012x over previous
import functools

import jax
import jax.numpy as jnp
from jax import lax
from jax.experimental import pallas as pl
from jax.experimental.pallas import tpu as pltpu
from jax.experimental.pallas import tpu_sc as plsc

_NEG = -1e30


def _sc_gather(emb, idx):
    (N,) = idx.shape
    _, D = emb.shape
    info = plsc.get_sparse_core_info()
    nw = info.num_cores * info.num_subcores
    ch = 128
    per_w = N // nw
    n_ch = per_w // ch
    mesh = plsc.VectorSubcoreMesh(core_axis_name="c", subcore_axis_name="s")

    @functools.partial(
        pl.kernel,
        mesh=mesh,
        compiler_params=pltpu.CompilerParams(use_tc_tiling_on_sc=False),
        out_type=jax.ShapeDtypeStruct((N, D), jnp.float32),
        scratch_types=[
            pltpu.VMEM((ch,), jnp.int32),
            pltpu.VMEM((ch, D), jnp.float32),
            pltpu.SemaphoreType.DMA,
        ],
    )
    def gk(emb_hbm, idx_hbm, out_hbm, idx_v, rows_v, sem):
        wid = lax.axis_index("s") * info.num_cores + lax.axis_index("c")
        base = wid * per_w

        def body(i, carry):
            off = base + i * ch
            pltpu.sync_copy(idx_hbm.at[pl.ds(off, ch)], idx_v)
            pltpu.async_copy(emb_hbm.at[idx_v], rows_v, sem).wait()
            pltpu.sync_copy(rows_v, out_hbm.at[pl.ds(off, ch)])
            return carry

        lax.fori_loop(0, n_ch, body, 0)

    return gk(emb, idx)


def _mlp(x, W1, b1, W2, b2, W3, b3, W4, b4):
    Bn, K = x.shape
    Hn = W1.shape[1]
    RB = 512

    def body(x_ref, w1r, b1r, w2r, b2r, w3r, b3r, w4r, b4r, o_ref):
        h = x_ref[...].astype(jnp.bfloat16)
        for w_ref, b_ref in ((w1r, b1r), (w2r, b2r), (w3r, b3r), (w4r, b4r)):
            z = jnp.dot(h, w_ref[...].astype(jnp.bfloat16),
                        preferred_element_type=jnp.float32)
            h = jnp.maximum(z + b_ref[...], 0.0).astype(jnp.bfloat16)
        o_ref[...] = h

    return pl.pallas_call(
        body,
        grid=(Bn // RB,),
        in_specs=[
            pl.BlockSpec((RB, K), lambda i: (i, 0)),
            pl.BlockSpec((K, Hn), lambda i: (0, 0)),
            pl.BlockSpec((1, Hn), lambda i: (0, 0)),
            pl.BlockSpec((Hn, Hn), lambda i: (0, 0)),
            pl.BlockSpec((1, Hn), lambda i: (0, 0)),
            pl.BlockSpec((Hn, Hn), lambda i: (0, 0)),
            pl.BlockSpec((1, Hn), lambda i: (0, 0)),
            pl.BlockSpec((Hn, Hn), lambda i: (0, 0)),
            pl.BlockSpec((1, Hn), lambda i: (0, 0)),
        ],
        out_specs=pl.BlockSpec((RB, Hn), lambda i: (i, 0)),
        out_shape=jax.ShapeDtypeStruct((Bn, Hn), jnp.bfloat16),
    )(x, W1, b1, W2, b2, W3, b3, W4, b4)


_TV = 1024
_ACC = 512
_TVP = 4096
_BTP = 1024


def _lse(h, W5, b5):
    Bn, Hn = h.shape
    V = W5.shape[1]
    nv = pl.cdiv(V, _TV)

    def body(h_ref, w_ref, b_ref, lse_ref, m_ref, acc_ref):
        v = pl.program_id(0)
        logits = jnp.dot(h_ref[...], w_ref[...].astype(jnp.bfloat16),
                         preferred_element_type=jnp.float32) + b_ref[...]

        @pl.when(v == 0)
        def _():
            m_ref[...] = jnp.max(logits, axis=1, keepdims=True)
            e = jnp.exp(logits - m_ref[...])
            acc_ref[...] = e[:, :_ACC] + e[:, _ACC:]

        @pl.when(jnp.logical_and(v > 0, v < nv - 1))
        def _():
            e = jnp.exp(logits - m_ref[...])
            acc_ref[...] = acc_ref[...] + e[:, :_ACC] + e[:, _ACC:]

        @pl.when(v == nv - 1)
        def _():
            cols = v * _TV + lax.broadcasted_iota(jnp.int32, (1, _TV), 1)
            e = jnp.exp(jnp.where(cols < V, logits - m_ref[...], _NEG))
            acc = acc_ref[...] + e[:, :_ACC] + e[:, _ACC:]
            lse_ref[...] = m_ref[...] + jnp.log(
                jnp.sum(acc, axis=1, keepdims=True))

    return pl.pallas_call(
        body,
        grid=(nv,),
        in_specs=[
            pl.BlockSpec((Bn, Hn), lambda v: (0, 0)),
            pl.BlockSpec((Hn, _TV), lambda v: (0, v)),
            pl.BlockSpec((1, _TV), lambda v: (0, v)),
        ],
        out_specs=pl.BlockSpec((Bn, 1), lambda v: (0, 0)),
        out_shape=jax.ShapeDtypeStruct((Bn, 1), jnp.float32),
        scratch_shapes=[
            pltpu.VMEM((Bn, 1), jnp.float32),
            pltpu.VMEM((Bn, _ACC), jnp.float32),
        ],
    )(h, W5, b5)


def _proj(h, W5, b5, lse):
    Bn, Hn = h.shape
    V = W5.shape[1]
    bt = min(_BTP, Bn)
    vp = pl.cdiv(V, 128) * 128
    nv = pl.cdiv(vp, _TVP)
    nb = Bn // bt

    def body(h_ref, w_ref, b_ref, lse_ref, o_ref):
        z = jnp.dot(h_ref[...], w_ref[...].astype(jnp.bfloat16),
                    preferred_element_type=jnp.float32)
        o_ref[...] = (z + b_ref[...] - lse_ref[...]).astype(jnp.bfloat16)

    return pl.pallas_call(
        body,
        grid=(nv, nb),
        in_specs=[
            pl.BlockSpec((bt, Hn), lambda v, b: (b, 0)),
            pl.BlockSpec((Hn, _TVP), lambda v, b: (0, v)),
            pl.BlockSpec((1, _TVP), lambda v, b: (0, v)),
            pl.BlockSpec((bt, 1), lambda v, b: (b, 0)),
        ],
        out_specs=pl.BlockSpec((bt, _TVP), lambda v, b: (b, v)),
        out_shape=jax.ShapeDtypeStruct((Bn, vp), jnp.bfloat16),
    )(h, W5, b5, lse)


def kernel(context_idxs, emb, W1, b1, W2, b2, W3, b3, W4, b4, W5, b5):
    Bn, C = context_idxs.shape
    _, D = emb.shape
    idx = context_idxs.reshape(-1).astype(jnp.int32)
    gathered = _sc_gather(emb, idx)
    x = gathered.reshape(Bn, C * D)
    h = _mlp(x, W1, b1.reshape(1, -1), W2, b2.reshape(1, -1),
             W3, b3.reshape(1, -1), W4, b4.reshape(1, -1))
    b5r = b5.reshape(1, -1)
    lse = _lse(h, W5, b5r)
    out_pad = _proj(h, W5, b5r, lse)
    V = W5.shape[1]
    return out_pad[:, :V].astype(jnp.float32)


# speedup vs baseline: 1.2615x; 1.2600x over previous
import functools

import jax
import jax.numpy as jnp
from jax import lax
from jax.experimental import pallas as pl
from jax.experimental.pallas import tpu as pltpu
from jax.experimental.pallas import tpu_sc as plsc

_NEG = -1e30


def _sc_gather(emb, idx):
    (N,) = idx.shape
    _, D = emb.shape
    info = plsc.get_sparse_core_info()
    nw = info.num_cores * info.num_subcores
    ch = 128
    per_w = N // nw
    n_ch = per_w // ch
    mesh = plsc.VectorSubcoreMesh(core_axis_name="c", subcore_axis_name="s")

    @functools.partial(
        pl.kernel,
        mesh=mesh,
        compiler_params=pltpu.CompilerParams(use_tc_tiling_on_sc=False),
        out_type=jax.ShapeDtypeStruct((N, D), jnp.float32),
        scratch_types=[
            pltpu.VMEM((ch,), jnp.int32),
            pltpu.VMEM((ch, D), jnp.float32),
            pltpu.SemaphoreType.DMA,
        ],
    )
    def gk(emb_hbm, idx_hbm, out_hbm, idx_v, rows_v, sem):
        wid = lax.axis_index("s") * info.num_cores + lax.axis_index("c")
        base = wid * per_w

        def body(i, carry):
            off = base + i * ch
            pltpu.sync_copy(idx_hbm.at[pl.ds(off, ch)], idx_v)
            pltpu.async_copy(emb_hbm.at[idx_v], rows_v, sem).wait()
            pltpu.sync_copy(rows_v, out_hbm.at[pl.ds(off, ch)])
            return carry

        lax.fori_loop(0, n_ch, body, 0)

    return gk(emb, idx)


def _mlp(x, W1, b1, W2, b2, W3, b3, W4, b4):
    Bn, K = x.shape
    Hn = W1.shape[1]
    RB = 512

    def body(x_ref, w1r, b1r, w2r, b2r, w3r, b3r, w4r, b4r, o_ref):
        h = x_ref[...].astype(jnp.bfloat16)
        for w_ref, b_ref in ((w1r, b1r), (w2r, b2r), (w3r, b3r), (w4r, b4r)):
            z = jnp.dot(h, w_ref[...].astype(jnp.bfloat16),
                        preferred_element_type=jnp.float32)
            h = jnp.maximum(z + b_ref[...], 0.0).astype(jnp.bfloat16)
        o_ref[...] = h

    return pl.pallas_call(
        body,
        grid=(Bn // RB,),
        in_specs=[
            pl.BlockSpec((RB, K), lambda i: (i, 0)),
            pl.BlockSpec((K, Hn), lambda i: (0, 0)),
            pl.BlockSpec((1, Hn), lambda i: (0, 0)),
            pl.BlockSpec((Hn, Hn), lambda i: (0, 0)),
            pl.BlockSpec((1, Hn), lambda i: (0, 0)),
            pl.BlockSpec((Hn, Hn), lambda i: (0, 0)),
            pl.BlockSpec((1, Hn), lambda i: (0, 0)),
            pl.BlockSpec((Hn, Hn), lambda i: (0, 0)),
            pl.BlockSpec((1, Hn), lambda i: (0, 0)),
        ],
        out_specs=pl.BlockSpec((RB, Hn), lambda i: (i, 0)),
        out_shape=jax.ShapeDtypeStruct((Bn, Hn), jnp.bfloat16),
    )(x, W1, b1, W2, b2, W3, b3, W4, b4)


_TV = 1024
_ACC = 512
_TVP = 4096
_BTP = 1024


def _lse(h, W5, b5):
    Bn, Hn = h.shape
    V = W5.shape[1]
    nv = pl.cdiv(V, _TV)

    def body(h_ref, w_ref, b_ref, lse_ref, m_ref, acc_ref):
        v = pl.program_id(0)
        logits = jnp.dot(h_ref[...], w_ref[...].astype(jnp.bfloat16),
                         preferred_element_type=jnp.float32) + b_ref[...]

        @pl.when(v == 0)
        def _():
            m_ref[...] = jnp.max(logits, axis=1, keepdims=True)
            e = jnp.exp(logits - m_ref[...])
            acc_ref[...] = e[:, :_ACC] + e[:, _ACC:]

        @pl.when(jnp.logical_and(v > 0, v < nv - 1))
        def _():
            e = jnp.exp(logits - m_ref[...])
            acc_ref[...] = acc_ref[...] + e[:, :_ACC] + e[:, _ACC:]

        @pl.when(v == nv - 1)
        def _():
            cols = v * _TV + lax.broadcasted_iota(jnp.int32, (1, _TV), 1)
            e = jnp.exp(jnp.where(cols < V, logits - m_ref[...], _NEG))
            acc = acc_ref[...] + e[:, :_ACC] + e[:, _ACC:]
            lse_ref[...] = m_ref[...] + jnp.log(
                jnp.sum(acc, axis=1, keepdims=True))

    return pl.pallas_call(
        body,
        grid=(nv,),
        in_specs=[
            pl.BlockSpec((Bn, Hn), lambda v: (0, 0)),
            pl.BlockSpec((Hn, _TV), lambda v: (0, v)),
            pl.BlockSpec((1, _TV), lambda v: (0, v)),
        ],
        out_specs=pl.BlockSpec((Bn, 1), lambda v: (0, 0)),
        out_shape=jax.ShapeDtypeStruct((Bn, 1), jnp.float32),
        scratch_shapes=[
            pltpu.VMEM((Bn, 1), jnp.float32),
            pltpu.VMEM((Bn, _ACC), jnp.float32),
        ],
    )(h, W5, b5)


def _proj(h, W5, b5, lse):
    Bn, Hn = h.shape
    V = W5.shape[1]
    bt = min(_BTP, Bn)
    vp = pl.cdiv(V, 128) * 128
    nv = pl.cdiv(vp, _TVP)
    nb = Bn // bt

    def body(h_ref, w_ref, b_ref, lse_ref, o_ref):
        z = jnp.dot(h_ref[...], w_ref[...].astype(jnp.bfloat16),
                    preferred_element_type=jnp.float32)
        o_ref[...] = z + b_ref[...] - lse_ref[...]

    return pl.pallas_call(
        body,
        grid=(nv, nb),
        in_specs=[
            pl.BlockSpec((bt, Hn), lambda v, b: (b, 0)),
            pl.BlockSpec((Hn, _TVP), lambda v, b: (0, v)),
            pl.BlockSpec((1, _TVP), lambda v, b: (0, v)),
            pl.BlockSpec((bt, 1), lambda v, b: (b, 0)),
        ],
        out_specs=pl.BlockSpec((bt, _TVP), lambda v, b: (b, v)),
        out_shape=jax.ShapeDtypeStruct((Bn, vp), jnp.float32),
    )(h, W5, b5, lse)


def kernel(context_idxs, emb, W1, b1, W2, b2, W3, b3, W4, b4, W5, b5):
    Bn, C = context_idxs.shape
    _, D = emb.shape
    idx = context_idxs.reshape(-1).astype(jnp.int32)
    gathered = _sc_gather(emb, idx)
    x = gathered.reshape(Bn, C * D)
    h = _mlp(x, W1, b1.reshape(1, -1), W2, b2.reshape(1, -1),
             W3, b3.reshape(1, -1), W4, b4.reshape(1, -1))
    b5r = b5.reshape(1, -1)
    lse = _lse(h, W5, b5r)
    out_pad = _proj(h, W5, b5r, lse)
    V = W5.shape[1]
    return out_pad[:, :V]
